# baseline (speedup 1.0000x reference)
import functools

import jax
import jax.numpy as jnp
from jax import lax
from jax.experimental import pallas as pl
from jax.experimental.pallas import tpu as pltpu
from jax.experimental.pallas import tpu_sc as plsc

VOCAB = 1000000
EMBED = 64
BATCH = 4096
HIST = 200

NC = 2
NS = 16
NW = NC * NS
BPW = BATCH // NW
CHUNK = 100
CPB = HIST // CHUNK
NCHUNK = BPW * CPB
NBUF = 4
NLANE = EMBED // 16


def _sc_pool(x2, table):
    mesh = plsc.VectorSubcoreMesh(core_axis_name="c", subcore_axis_name="s")

    @functools.partial(
        pl.kernel,
        out_type=jax.ShapeDtypeStruct((BATCH, EMBED), jnp.float32),
        mesh=mesh,
        compiler_params=pltpu.CompilerParams(use_tc_tiling_on_sc=False),
        scratch_types=[
            pltpu.VMEM((NCHUNK, CHUNK), jnp.int32),
            pltpu.VMEM((NBUF, CHUNK, EMBED), jnp.float32),
            pltpu.VMEM((BPW, EMBED), jnp.float32),
            pltpu.SemaphoreType.DMA,
            pltpu.SemaphoreType.DMA,
            pltpu.SemaphoreType.DMA,
            pltpu.SemaphoreType.DMA,
        ],
    )
    def pool(x_hbm, table_hbm, out_hbm, idx_v, rows_v, out_v, s0, s1, s2, s3):
        sems = (s0, s1, s2, s3)
        wid = lax.axis_index("s") * NC + lax.axis_index("c")
        pltpu.sync_copy(x_hbm.at[pl.ds(wid * NCHUNK, NCHUNK)], idx_v)

        def start(chunk, b):
            pltpu.make_async_copy(
                table_hbm.at[idx_v.at[chunk]], rows_v.at[b], sems[b]
            ).start()

        def wait(chunk, b):
            pltpu.make_async_copy(
                table_hbm.at[idx_v.at[chunk]], rows_v.at[b], sems[b]
            ).wait()

        for b in range(NBUF):
            start(b, b)

        def reduce_chunk(b, acc):
            def body(r, acc):
                return tuple(
                    acc[c] + rows_v[b, r, pl.ds(16 * c, 16)]
                    for c in range(NLANE)
                )
            return lax.fori_loop(0, CHUNK, body, acc, unroll=4)

        zeros = tuple(jnp.zeros((16,), jnp.float32) for _ in range(NLANE))

        def group(i, do_issue):
            j = i * NBUF
            for rr in range(NBUF // CPB):
                acc = zeros
                for cc in range(CPB):
                    b = rr * CPB + cc
                    chunk = j + b
                    wait(chunk, b)
                    acc = reduce_chunk(b, acc)
                    if do_issue:
                        start(chunk + NBUF, b)
                row = i * (NBUF // CPB) + rr
                for c in range(NLANE):
                    out_v[row, pl.ds(16 * c, 16)] = acc[c]

        def loop_body(i, carry):
            group(i, True)
            return carry

        lax.fori_loop(0, NCHUNK // NBUF - 1, loop_body, 0)
        group(NCHUNK // NBUF - 1, False)

        pltpu.sync_copy(out_v, out_hbm.at[pl.ds(wid * BPW, BPW)])

    return pool(x2, table)


def _tc_finish_body(p_ref, w_ref, b_ref, bg_ref, bb_ref, lg_ref, lb_ref,
                    o_ref):
    p = p_ref[...] * (1.0 / HIST)
    h = lax.dot_general(
        p, w_ref[...], (((1,), (0,)), ((), ())),
        preferred_element_type=jnp.float32,
    )
    h = h + b_ref[...]
    m = jnp.mean(h, axis=0, keepdims=True)
    d = h - m
    v = jnp.mean(d * d, axis=0, keepdims=True)
    h = d * lax.rsqrt(v + 1e-5) * bg_ref[...] + bb_ref[...]
    lm = jnp.mean(h, axis=1, keepdims=True)
    ld = h - lm
    lv = jnp.mean(ld * ld, axis=1, keepdims=True)
    o_ref[...] = ld * lax.rsqrt(lv + 1e-5) * lg_ref[...] + lb_ref[...]


def _tc_finish(pooled, wt, b2, bg2, bb2, lg2, lb2):
    return pl.pallas_call(
        _tc_finish_body,
        out_shape=jax.ShapeDtypeStruct((BATCH, EMBED), jnp.float32),
    )(pooled, wt, b2, bg2, bb2, lg2, lb2)


def kernel(x, table, W, b, bn_gamma, bn_beta, ln_gamma, ln_beta):
    x2 = x.reshape(BATCH * CPB, CHUNK).astype(jnp.int32)
    pooled = _sc_pool(x2, table)
    r = lambda t: t.reshape(1, EMBED)
    return _tc_finish(pooled, W.T, r(b), r(bn_gamma), r(bn_beta),
                      r(ln_gamma), r(ln_beta))


# speedup vs baseline: 1.8014x; 1.8014x over previous
import functools

import jax
import jax.numpy as jnp
from jax import lax
from jax.experimental import pallas as pl
from jax.experimental.pallas import tpu as pltpu
from jax.experimental.pallas import tpu_sc as plsc

VOCAB = 1000000
EMBED = 64
BATCH = 4096
HIST = 200

NC = 2
NS = 16
NW = NC * NS
BPW = BATCH // NW
CHUNK = 100
CPB = HIST // CHUNK
NCHUNK = BPW * CPB
NBUF = 4
NACC = EMBED // 16

_QBLK = 6144
_RGRID = -(-VOCAB // (4 * _QBLK))
TROWS = 4 * _QBLK * _RGRID
_LASTBLK = (VOCAB - 1) // _QBLK
EM2 = EMBED // 2

import numpy as np

_FEATURE_PERM = np.concatenate(
    [np.arange(0, 32, 2), np.arange(1, 32, 2),
     np.arange(32, 64, 2), np.arange(33, 64, 2)])

_HIMASK = -65536


def _round_to_bf16_bits(x):
    u = lax.bitcast_convert_type(x, jnp.int32)
    return (u + 0x7FFF + ((u >> 16) & 1)) >> 16


def _repack_body(a_ref, b_ref, c_ref, d_ref, o_ref):
    for q, ref in enumerate((a_ref, b_ref, c_ref, d_ref)):
        bits = _round_to_bf16_bits(ref[...])
        b3 = bits.reshape(EM2, 2, _QBLK)
        packed = (b3[:, 0, :] & 0xFFFF) | (b3[:, 1, :] << 16)
        o_ref[:, q * EM2:(q + 1) * EM2] = packed.T


def _repack(table_t):
    return pl.pallas_call(
        _repack_body,
        grid=(_RGRID,),
        in_specs=[
            pl.BlockSpec(
                (EMBED, _QBLK),
                lambda i, q=q: (0, jnp.minimum(4 * i + q, _LASTBLK)))
            for q in range(4)
        ],
        out_specs=pl.BlockSpec((_QBLK, 4 * EM2), lambda i: (i, 0)),
        out_shape=jax.ShapeDtypeStruct((TROWS // 4, 4 * EM2), jnp.int32),
    )(table_t, table_t, table_t, table_t)


def _sc_pool(x2, tableb):
    mesh = plsc.VectorSubcoreMesh(core_axis_name="c", subcore_axis_name="s")

    @functools.partial(
        pl.kernel,
        out_type=jax.ShapeDtypeStruct((BATCH, EMBED), jnp.float32),
        mesh=mesh,
        compiler_params=pltpu.CompilerParams(
            use_tc_tiling_on_sc=False, needs_layout_passes=False),
        scratch_types=[
            pltpu.VMEM((NCHUNK, CHUNK), jnp.int32),
            pltpu.VMEM((NBUF, CHUNK, EM2), jnp.int32),
            pltpu.VMEM((BPW, EMBED), jnp.float32),
            pltpu.SemaphoreType.DMA,
            pltpu.SemaphoreType.DMA,
            pltpu.SemaphoreType.DMA,
            pltpu.SemaphoreType.DMA,
        ],
    )
    def pool(x_hbm, table_hbm, out_hbm, idx_v, rows_v, out_v, s0, s1, s2, s3):
        sems = (s0, s1, s2, s3)
        wid = lax.axis_index("s") * NC + lax.axis_index("c")
        pltpu.sync_copy(x_hbm.at[pl.ds(wid * NCHUNK, NCHUNK)], idx_v)

        def start(chunk, b):
            pltpu.make_async_copy(
                table_hbm.at[idx_v.at[chunk]], rows_v.at[b], sems[b]
            ).start()

        def wait(chunk, b):
            pltpu.make_async_copy(
                table_hbm.at[idx_v.at[chunk]], rows_v.at[b], sems[b]
            ).wait()

        for b in range(NBUF):
            start(b, b)

        def reduce_chunk(b, acc):
            def body(r, acc):
                a0, a1, a2, a3 = acc
                v0 = rows_v[b, r, pl.ds(0, 16)]
                v1 = rows_v[b, r, pl.ds(16, 16)]
                a0 = a0 + plsc.bitcast(v0 << 16, jnp.float32)
                a1 = a1 + plsc.bitcast(v0 & _HIMASK, jnp.float32)
                a2 = a2 + plsc.bitcast(v1 << 16, jnp.float32)
                a3 = a3 + plsc.bitcast(v1 & _HIMASK, jnp.float32)
                return (a0, a1, a2, a3)
            return lax.fori_loop(0, CHUNK, body, acc, unroll=4)

        zeros = tuple(jnp.zeros((16,), jnp.float32) for _ in range(NACC))

        def group(i, do_issue):
            j = i * NBUF
            for rr in range(NBUF // CPB):
                acc = zeros
                for cc in range(CPB):
                    b = rr * CPB + cc
                    chunk = j + b
                    wait(chunk, b)
                    acc = reduce_chunk(b, acc)
                    if do_issue:
                        start(chunk + NBUF, b)
                row = i * (NBUF // CPB) + rr
                for c in range(NACC):
                    out_v[row, pl.ds(16 * c, 16)] = acc[c]

        def loop_body(i, carry):
            group(i, True)
            return carry

        lax.fori_loop(0, NCHUNK // NBUF - 1, loop_body, 0)
        group(NCHUNK // NBUF - 1, False)

        pltpu.sync_copy(out_v, out_hbm.at[pl.ds(wid * BPW, BPW)])

    return pool(x2, tableb)


def _tc_finish_body(p_ref, w_ref, b_ref, bg_ref, bb_ref, lg_ref, lb_ref,
                    o_ref):
    p = p_ref[...] * (1.0 / HIST)
    h = lax.dot_general(
        p, w_ref[...], (((1,), (0,)), ((), ())),
        preferred_element_type=jnp.float32,
    )
    h = h + b_ref[...]
    m = jnp.mean(h, axis=0, keepdims=True)
    d = h - m
    v = jnp.mean(d * d, axis=0, keepdims=True)
    h = d * lax.rsqrt(v + 1e-5) * bg_ref[...] + bb_ref[...]
    lm = jnp.mean(h, axis=1, keepdims=True)
    ld = h - lm
    lv = jnp.mean(ld * ld, axis=1, keepdims=True)
    o_ref[...] = ld * lax.rsqrt(lv + 1e-5) * lg_ref[...] + lb_ref[...]


def _tc_finish(pooled, wt, b2, bg2, bb2, lg2, lb2):
    return pl.pallas_call(
        _tc_finish_body,
        out_shape=jax.ShapeDtypeStruct((BATCH, EMBED), jnp.float32),
    )(pooled, wt, b2, bg2, bb2, lg2, lb2)


def kernel(x, table, W, b, bn_gamma, bn_beta, ln_gamma, ln_beta):
    xi = x.astype(jnp.int32)
    span = 4 * _QBLK
    off = xi % span
    q = off // _QBLK
    o = off % _QBLK
    xv = (xi - off) + (o << 2) + q
    x2 = xv.reshape(BATCH * CPB, CHUNK)
    tb = _repack(table.T).reshape(TROWS, EM2)
    pooled = _sc_pool(x2, tb)
    wt = W.T[_FEATURE_PERM, :]
    r = lambda t: t.reshape(1, EMBED)
    return _tc_finish(pooled, wt, r(b), r(bn_gamma), r(bn_beta),
                      r(ln_gamma), r(ln_beta))


# speedup vs baseline: 2.0831x; 1.1564x over previous
import functools

import jax
import jax.numpy as jnp
from jax import lax
from jax.experimental import pallas as pl
from jax.experimental.pallas import tpu as pltpu
from jax.experimental.pallas import tpu_sc as plsc

VOCAB = 1000000
EMBED = 64
BATCH = 4096
HIST = 200

NC = 2
NS = 16
NW = NC * NS
BPW = BATCH // NW
CHUNK = 100
CPB = HIST // CHUNK
NCHUNK = BPW * CPB
NBUF = 4
NACC = EMBED // 16

_HBLK = 12288
_RGRID = -(-VOCAB // (2 * _HBLK))
TROWS = 2 * _HBLK * _RGRID
_LASTBLK = (VOCAB - 1) // _HBLK


def _repack_body(a_ref, b_ref, o_ref):
    o_ref[:, 0:EMBED] = a_ref[...].T
    o_ref[:, EMBED:2 * EMBED] = b_ref[...].T


def _repack(table_t):
    return pl.pallas_call(
        _repack_body,
        grid=(_RGRID,),
        in_specs=[
            pl.BlockSpec((EMBED, _HBLK),
                         lambda i: (0, jnp.minimum(2 * i, _LASTBLK))),
            pl.BlockSpec((EMBED, _HBLK),
                         lambda i: (0, jnp.minimum(2 * i + 1, _LASTBLK))),
        ],
        out_specs=pl.BlockSpec((_HBLK, 2 * EMBED), lambda i: (i, 0)),
        out_shape=jax.ShapeDtypeStruct((TROWS // 2, 2 * EMBED), jnp.float32),
    )(table_t, table_t)


def _sc_pool(x2, tableb):
    mesh = plsc.VectorSubcoreMesh(core_axis_name="c", subcore_axis_name="s")

    @functools.partial(
        pl.kernel,
        out_type=jax.ShapeDtypeStruct((BATCH, EMBED), jnp.float32),
        mesh=mesh,
        compiler_params=pltpu.CompilerParams(use_tc_tiling_on_sc=False),
        scratch_types=[
            pltpu.VMEM((NCHUNK, CHUNK), jnp.int32),
            pltpu.VMEM((NBUF, CHUNK, EMBED), jnp.float32),
            pltpu.VMEM((BPW, EMBED), jnp.float32),
            pltpu.SemaphoreType.DMA,
            pltpu.SemaphoreType.DMA,
            pltpu.SemaphoreType.DMA,
            pltpu.SemaphoreType.DMA,
        ],
    )
    def pool(x_hbm, table_hbm, out_hbm, idx_v, rows_v, out_v, s0, s1, s2, s3):
        sems = (s0, s1, s2, s3)
        wid = lax.axis_index("s") * NC + lax.axis_index("c")
        pltpu.sync_copy(x_hbm.at[pl.ds(wid * NCHUNK, NCHUNK)], idx_v)

        def start(chunk, b):
            pltpu.make_async_copy(
                table_hbm.at[idx_v.at[chunk]], rows_v.at[b], sems[b]
            ).start()

        def wait(chunk, b):
            pltpu.make_async_copy(
                table_hbm.at[idx_v.at[chunk]], rows_v.at[b], sems[b]
            ).wait()

        for b in range(NBUF):
            start(b, b)

        def reduce_chunk(b, acc):
            def body(r, acc):
                return tuple(
                    acc[c] + rows_v[b, r, pl.ds(16 * c, 16)]
                    for c in range(NACC)
                )
            return lax.fori_loop(0, CHUNK, body, acc, unroll=4)

        zeros = tuple(jnp.zeros((16,), jnp.float32) for _ in range(NACC))

        def group(i, do_issue):
            j = i * NBUF
            for rr in range(NBUF // CPB):
                acc = zeros
                for cc in range(CPB):
                    b = rr * CPB + cc
                    chunk = j + b
                    wait(chunk, b)
                    acc = reduce_chunk(b, acc)
                    if do_issue:
                        start(chunk + NBUF, b)
                row = i * (NBUF // CPB) + rr
                for c in range(NACC):
                    out_v[row, pl.ds(16 * c, 16)] = acc[c]

        def loop_body(i, carry):
            group(i, True)
            return carry

        lax.fori_loop(0, NCHUNK // NBUF - 1, loop_body, 0)
        group(NCHUNK // NBUF - 1, False)

        pltpu.sync_copy(out_v, out_hbm.at[pl.ds(wid * BPW, BPW)])

    return pool(x2, tableb)


def _tc_finish_body(p_ref, w_ref, b_ref, bg_ref, bb_ref, lg_ref, lb_ref,
                    o_ref):
    p = p_ref[...] * (1.0 / HIST)
    h = lax.dot_general(
        p, w_ref[...], (((1,), (0,)), ((), ())),
        preferred_element_type=jnp.float32,
    )
    h = h + b_ref[...]
    m = jnp.mean(h, axis=0, keepdims=True)
    d = h - m
    v = jnp.mean(d * d, axis=0, keepdims=True)
    h = d * lax.rsqrt(v + 1e-5) * bg_ref[...] + bb_ref[...]
    lm = jnp.mean(h, axis=1, keepdims=True)
    ld = h - lm
    lv = jnp.mean(ld * ld, axis=1, keepdims=True)
    o_ref[...] = ld * lax.rsqrt(lv + 1e-5) * lg_ref[...] + lb_ref[...]


def _tc_finish(pooled, wt, b2, bg2, bb2, lg2, lb2):
    return pl.pallas_call(
        _tc_finish_body,
        out_shape=jax.ShapeDtypeStruct((BATCH, EMBED), jnp.float32),
    )(pooled, wt, b2, bg2, bb2, lg2, lb2)


def kernel(x, table, W, b, bn_gamma, bn_beta, ln_gamma, ln_beta):
    xi = x.astype(jnp.int32)
    span = 2 * _HBLK
    off = xi % span
    xv = (xi - off) + jnp.where(off < _HBLK, off << 1,
                                ((off - _HBLK) << 1) + 1)
    x2 = xv.reshape(BATCH * CPB, CHUNK)
    tb = _repack(table.T).reshape(TROWS, EMBED)
    pooled = _sc_pool(x2, tb)
    r = lambda t: t.reshape(1, EMBED)
    return _tc_finish(pooled, W.T, r(b), r(bn_gamma), r(bn_beta),
                      r(ln_gamma), r(ln_beta))
